# baseline (speedup 1.0000x reference)
import functools

import jax
import jax.numpy as jnp
from jax import lax
from jax.experimental import pallas as pl
from jax.experimental.pallas import tpu as pltpu
from jax.experimental.pallas import tpu_sc as plsc

N = 16384
K = 50
KPAD = 64
NUM_BINS = 5
MAX_DIST = 10.0
GAMMA = 0.5

NC = 2
NS = 16
NW = NC * NS
QPW = N // NW
NCHUNK = N // 16
NSUB = 4096
BUFCAP = 4096
STGQ = 32
HIST_FLAT = False


def _knn_sc_body(x_h, y_h, z_h, nbr_out, d2_out,
                 x_v, y_v, z_v, bufb, bufi, hist2, tot_v,
                 outb, outi, stg_i, stg_d, sem):
    lane = lax.iota(jnp.int32, 16)
    lane256 = lane * 256
    zeros16 = jnp.zeros((16,), jnp.int32)
    ones16 = jnp.ones((16,), jnp.int32)
    negones16 = -ones16
    inf_bits = jnp.full((16,), 0x7F800000, jnp.int32)
    nhist = 16 if HIST_FLAT else 256

    def histidx(key):
        return key if HIST_FLAT else lane256 + key

    def scan_first_bin(need):
        def g(gi, carry):
            cum, nb_v = carry
            if HIST_FLAT:
                acc = hist2[pl.ds(gi * 16, 16)]
            else:
                acc = zeros16
                for l in range(16):
                    acc = acc + hist2[pl.ds(l * 256 + gi * 16, 16)]
            tot_v[pl.ds(gi * 16, 16)] = acc
            cs = plsc.cumsum(acc) + cum
            nb_v = nb_v + plsc.all_reduce_population_count(cs < need)
            return jnp.max(cs), nb_v

        _, nb_v = lax.fori_loop(0, 16, g, (jnp.int32(0), zeros16))
        return jnp.max(nb_v)

    def count_below(b):
        def g(gi, cb_v):
            v = tot_v[pl.ds(gi * 16, 16)]
            return cb_v + jnp.where(gi * 16 + lane < b, v, 0)

        return jnp.sum(lax.fori_loop(0, 16, g, zeros16))

    wid = lax.axis_index("s") * NC + lax.axis_index("c")
    pltpu.sync_copy(x_h, x_v)
    pltpu.sync_copy(y_h, y_v)
    pltpu.sync_copy(z_h, z_v)

    @plsc.parallel_loop(0, nhist, unroll=4)
    def _clr(i):
        hist2[pl.ds(i * 16, 16)] = zeros16

    def per_query(qi, _):
        q = wid * QPW + qi
        qsplat = jnp.full((16,), q)
        qx = plsc.load_gather(x_v, [qsplat])
        qy = plsc.load_gather(y_v, [qsplat])
        qz = plsc.load_gather(z_v, [qsplat])

        def dist2(base):
            dx = x_v[pl.ds(base, 16)] - qx
            dy = y_v[pl.ds(base, 16)] - qy
            dz = z_v[pl.ds(base, 16)] - qz
            d2 = dx * dx + dy * dy + dz * dz
            return lax.bitcast_convert_type(d2, jnp.int32)

        def sub_hist(c, delta):
            bits = dist2(c * 16)
            key = lax.shift_right_logical(bits, 23)
            plsc.addupdate_scatter(hist2, [histidx(key)], delta)

        plsc.parallel_loop(0, NSUB // 16, unroll=8)(
            functools.partial(sub_hist, delta=ones16))
        b1s = scan_first_bin(K + 1)
        hi = lax.shift_left(jnp.minimum(b1s + 1, 255), 23)
        hi_splat = jnp.full((16,), hi)

        @plsc.parallel_loop(0, nhist, unroll=8)
        def _clr_q(i):
            hist2[pl.ds(i * 16, 16)] = zeros16

        @plsc.parallel_loop(0, NCHUNK, unroll=2, carry=zeros16)
        def mainsweep(c, cnt_v):
            base = c * 16
            bits = dist2(base)
            idxv = base + lane
            keep = (bits < hi_splat) & (idxv != qsplat)
            pos = cnt_v + plsc.cumsum(jnp.where(keep, 1, 0)) - 1
            okm = keep & (pos < BUFCAP)
            plsc.store_scatter(bufb, [pos], bits, mask=okm)
            plsc.store_scatter(bufi, [pos], idxv, mask=okm)
            return cnt_v + plsc.all_reduce_population_count(keep)

        m_tot = jnp.max(mainsweep)
        nch = lax.shift_right_logical(m_tot + 15, 4)

        lo = jnp.int32(0)
        cb = jnp.int32(0)
        for shift in (23, 15, 7):
            need = K - cb
            lo_s = jnp.full((16,), lo)
            hi_s = jnp.full((16,), hi)

            def lvl(c, delta, lo_s=lo_s, hi_s=hi_s, shift=shift):
                b = bufb[pl.ds(c * 16, 16)]
                valid = (c * 16 + lane) < m_tot
                inr = (b >= lo_s) & (b < hi_s) & valid
                key = jnp.bitwise_and(lax.shift_right_logical(b, shift), 255)
                plsc.addupdate_scatter(hist2, [histidx(key)], delta, mask=inr)

            plsc.parallel_loop(0, nch, unroll=4)(
                functools.partial(lvl, delta=ones16))
            b2 = scan_first_bin(need)
            cb2 = count_below(b2)
            plsc.parallel_loop(0, nch, unroll=4)(
                functools.partial(lvl, delta=negones16))
            lo = lo + lax.shift_left(b2, shift)
            hi = lo + lax.shift_left(jnp.int32(1), shift)
            cb = cb + cb2

        for j in range(KPAD // 16):
            outb[pl.ds(j * 16, 16)] = inf_bits
            outi[pl.ds(j * 16, 16)] = zeros16

        lo_s = jnp.full((16,), lo)
        hi_s = jnp.full((16,), hi)

        @plsc.parallel_loop(0, nch, unroll=2, carry=(zeros16, zeros16))
        def coll(c, carry):
            cs_v, cm_v = carry
            b = bufb[pl.ds(c * 16, 16)]
            ix = bufi[pl.ds(c * 16, 16)]
            valid = (c * 16 + lane) < m_tot
            sure = (b < lo_s) & valid
            mid = (b >= lo_s) & (b < hi_s) & valid
            psure = cs_v + plsc.cumsum(jnp.where(sure, 1, 0)) - 1
            pmid = cb + cm_v + plsc.cumsum(jnp.where(mid, 1, 0)) - 1
            okm = mid & (pmid < K)
            plsc.store_scatter(outb, [psure], b, mask=sure)
            plsc.store_scatter(outi, [psure], ix, mask=sure)
            plsc.store_scatter(outb, [pmid], b, mask=okm)
            plsc.store_scatter(outi, [pmid], ix, mask=okm)
            return (cs_v + plsc.all_reduce_population_count(sure),
                    cm_v + plsc.all_reduce_population_count(mid))

        def minmax(ak, av, bk, bv):
            c = ak <= bk
            return (jnp.where(c, ak, bk), jnp.where(c, av, bv),
                    jnp.where(c, bk, ak), jnp.where(c, bv, av))

        def merge16(ak, av, bk, bv):
            rk = lax.rev(bk, (0,))
            rv = lax.rev(bv, (0,))
            lk, lv, hk, hv = minmax(ak, av, rk, rv)
            lk, lv = plsc.sort_key_val(lk, lv)
            hk, hv = plsc.sort_key_val(hk, hv)
            return lk, lv, hk, hv

        sk, sv = [], []
        for j in range(4):
            kj, vj = plsc.sort_key_val(outb[pl.ds(j * 16, 16)],
                                       outi[pl.ds(j * 16, 16)])
            sk.append(kj)
            sv.append(vj)
        a0k, a0v, a1k, a1v = merge16(sk[0], sv[0], sk[1], sv[1])
        b0k, b0v, b1k, b1v = merge16(sk[2], sv[2], sk[3], sv[3])
        rb0k, rb0v = lax.rev(b1k, (0,)), lax.rev(b1v, (0,))
        rb1k, rb1v = lax.rev(b0k, (0,)), lax.rev(b0v, (0,))
        l0k, l0v, h0k, h0v = minmax(a0k, a0v, rb0k, rb0v)
        l1k, l1v, h1k, h1v = minmax(a1k, a1v, rb1k, rb1v)
        p0k, p0v, p1k, p1v = minmax(l0k, l0v, l1k, l1v)
        q0k, q0v, q1k, q1v = minmax(h0k, h0v, h1k, h1v)
        f0k, f0v = plsc.sort_key_val(p0k, p0v)
        f1k, f1v = plsc.sort_key_val(p1k, p1v)
        f2k, f2v = plsc.sort_key_val(q0k, q0v)
        f3k, f3v = plsc.sort_key_val(q1k, q1v)

        sbase = jnp.bitwise_and(qi, STGQ - 1) * K
        for j, (fk, fv) in enumerate(((f0k, f0v), (f1k, f1v),
                                      (f2k, f2v))):
            spos = sbase + j * 16 + lane
            plsc.store_scatter(stg_i, [spos], fv)
            plsc.store_scatter(stg_d, [spos],
                               lax.bitcast_convert_type(fk, jnp.float32))
        tail = lane < 2
        tpos = sbase + 48 + lane
        plsc.store_scatter(stg_i, [tpos], f3v, mask=tail)
        plsc.store_scatter(stg_d, [tpos],
                           lax.bitcast_convert_type(f3k, jnp.float32),
                           mask=tail)

        @pl.when(jnp.bitwise_and(qi, STGQ - 1) == STGQ - 1)
        def _flush():
            hbase = pl.multiple_of((q - (STGQ - 1)) * K, STGQ * K)
            pltpu.sync_copy(stg_i, nbr_out.at[pl.ds(hbase, STGQ * K)])
            pltpu.sync_copy(stg_d, d2_out.at[pl.ds(hbase, STGQ * K)])

        return 0

    lax.fori_loop(0, QPW, per_query, 0)


@functools.partial(
    pl.kernel,
    out_type=(jax.ShapeDtypeStruct((N * K,), jnp.int32),
              jax.ShapeDtypeStruct((N * K,), jnp.float32)),
    mesh=plsc.VectorSubcoreMesh(core_axis_name="c", subcore_axis_name="s"),
    compiler_params=pltpu.CompilerParams(needs_layout_passes=False),
    scratch_types=[
        pltpu.VMEM((N,), jnp.float32),
        pltpu.VMEM((N,), jnp.float32),
        pltpu.VMEM((N,), jnp.float32),
        pltpu.VMEM((BUFCAP,), jnp.int32),
        pltpu.VMEM((BUFCAP,), jnp.int32),
        pltpu.VMEM((4096,), jnp.int32),
        pltpu.VMEM((256,), jnp.int32),
        pltpu.VMEM((KPAD,), jnp.int32),
        pltpu.VMEM((KPAD,), jnp.int32),
        pltpu.VMEM((STGQ * K,), jnp.int32),
        pltpu.VMEM((STGQ * K,), jnp.float32),
        pltpu.SemaphoreType.DMA,
    ],
)
def _knn_sc(x_h, y_h, z_h, nbr_out, d2_out, *rest):
    _knn_sc_body(x_h, y_h, z_h, nbr_out, d2_out, *rest)


_RDF_BLK = 8192


def _rdf_kernel(d2_ref, out_ref):
    r = jnp.sqrt(d2_ref[...])
    mus = [MAX_DIST * i / (NUM_BINS - 1) for i in range(NUM_BINS)]
    rows = [jnp.exp(-GAMMA * (r - m) ** 2)[None, :] for m in mus]
    out_ref[...] = jnp.concatenate(rows, axis=0)


def _rdf5(d2):
    e = d2.shape[0]
    nblk = e // _RDF_BLK
    return pl.pallas_call(
        _rdf_kernel,
        grid=(2 * nblk,),
        in_specs=[pl.BlockSpec((_RDF_BLK,), lambda i: (i % nblk,))],
        out_specs=pl.BlockSpec((NUM_BINS, _RDF_BLK), lambda i: (0, i)),
        out_shape=jax.ShapeDtypeStruct((NUM_BINS, 2 * e), jnp.float32),
    )(d2)


_EI_ROWS = N * K // 128
_EI_RB = 400


def _edge_kernel(src_ref, out_ref):
    j = pl.program_id(0)
    s = src_ref[...]
    r_io = lax.broadcasted_iota(jnp.int32, (_EI_RB, 128), 0)
    c_io = lax.broadcasted_iota(jnp.int32, (_EI_RB, 128), 1)
    e = (j * _EI_RB + r_io) * 128 + c_io
    d = e // K
    out_ref[0, 0] = s
    out_ref[0, 1] = d
    out_ref[1, 0] = d
    out_ref[1, 1] = s


def _edge_index(src):
    out = pl.pallas_call(
        _edge_kernel,
        grid=(_EI_ROWS // _EI_RB,),
        in_specs=[pl.BlockSpec((_EI_RB, 128), lambda j: (j, 0))],
        out_specs=pl.BlockSpec((2, 2, _EI_RB, 128), lambda j: (0, 0, j, 0)),
        out_shape=jax.ShapeDtypeStruct((2, 2, _EI_ROWS, 128), jnp.int32),
    )(src.reshape(_EI_ROWS, 128))
    return out.reshape(2, 2 * N * K)


def kernel(pos):
    nbr50, d2_50 = _knn_sc(pos[:, 0], pos[:, 1], pos[:, 2])
    rdf = _rdf5(d2_50).T
    edge_index = _edge_index(nbr50)
    return edge_index, rdf


# speedup vs baseline: 1.0164x; 1.0164x over previous
import functools

import jax
import jax.numpy as jnp
from jax import lax
from jax.experimental import pallas as pl
from jax.experimental.pallas import tpu as pltpu
from jax.experimental.pallas import tpu_sc as plsc

N = 16384
K = 50
KPAD = 64
NUM_BINS = 5
MAX_DIST = 10.0
GAMMA = 0.5

NC = 2
NS = 16
NW = NC * NS
QPW = N // NW
NCHUNK = N // 16
NSUB = 4096
BUFCAP = 4096
STGQ = 32
HIST_FLAT = False


def _knn_sc_body(x_h, y_h, z_h, nbr_out, d2_out,
                 x_v, y_v, z_v, bufb, bufi, hist2, tot_v, bitsc,
                 outb, outi, stg_i, stg_d, sem):
    lane = lax.iota(jnp.int32, 16)
    lane256 = lane * 256
    zeros16 = jnp.zeros((16,), jnp.int32)
    ones16 = jnp.ones((16,), jnp.int32)
    negones16 = -ones16
    inf_bits = jnp.full((16,), 0x7F800000, jnp.int32)
    nhist = 16 if HIST_FLAT else 256

    def histidx(key):
        return key if HIST_FLAT else lane256 + key

    def scan_first_bin(need):
        def g(gi, carry):
            cum, nb_v = carry
            if HIST_FLAT:
                acc = hist2[pl.ds(gi * 16, 16)]
            else:
                acc = zeros16
                for l in range(16):
                    acc = acc + hist2[pl.ds(l * 256 + gi * 16, 16)]
            tot_v[pl.ds(gi * 16, 16)] = acc
            cs = plsc.cumsum(acc) + cum
            nb_v = nb_v + plsc.all_reduce_population_count(cs < need)
            return jnp.max(cs), nb_v

        _, nb_v = lax.fori_loop(0, 16, g, (jnp.int32(0), zeros16))
        return jnp.max(nb_v)

    def count_below(b):
        def g(gi, cb_v):
            v = tot_v[pl.ds(gi * 16, 16)]
            return cb_v + jnp.where(gi * 16 + lane < b, v, 0)

        return jnp.sum(lax.fori_loop(0, 16, g, zeros16))

    wid = lax.axis_index("s") * NC + lax.axis_index("c")
    pltpu.sync_copy(x_h, x_v)
    pltpu.sync_copy(y_h, y_v)
    pltpu.sync_copy(z_h, z_v)

    @plsc.parallel_loop(0, nhist, unroll=4)
    def _clr(i):
        hist2[pl.ds(i * 16, 16)] = zeros16

    def per_query(qi, _):
        q = wid * QPW + qi
        qsplat = jnp.full((16,), q)
        qx = plsc.load_gather(x_v, [qsplat])
        qy = plsc.load_gather(y_v, [qsplat])
        qz = plsc.load_gather(z_v, [qsplat])

        def dist2(base):
            dx = x_v[pl.ds(base, 16)] - qx
            dy = y_v[pl.ds(base, 16)] - qy
            dz = z_v[pl.ds(base, 16)] - qz
            d2 = dx * dx + dy * dy + dz * dz
            return lax.bitcast_convert_type(d2, jnp.int32)

        @plsc.parallel_loop(0, NSUB // 16, unroll=8)
        def _sub_hist(c):
            bits = dist2(c * 16)
            bitsc[pl.ds(c * 16, 16)] = bits
            key = lax.shift_right_logical(bits, 23)
            plsc.addupdate_scatter(hist2, [histidx(key)], ones16)
        b1s = scan_first_bin(K + 1)
        hi = lax.shift_left(jnp.minimum(b1s + 1, 255), 23)
        hi_splat = jnp.full((16,), hi)

        @plsc.parallel_loop(0, nhist, unroll=8)
        def _clr_q(i):
            hist2[pl.ds(i * 16, 16)] = zeros16

        def collect(bits, base, cnt_v):
            idxv = base + lane
            keep = (bits < hi_splat) & (idxv != qsplat)
            pos = cnt_v + plsc.cumsum(jnp.where(keep, 1, 0)) - 1
            okm = keep & (pos < BUFCAP)
            plsc.store_scatter(bufb, [pos], bits, mask=okm)
            plsc.store_scatter(bufi, [pos], idxv, mask=okm)
            return cnt_v + plsc.all_reduce_population_count(keep)

        @plsc.parallel_loop(0, NSUB // 16, unroll=4, carry=zeros16)
        def sweep_a(c, cnt_v):
            base = c * 16
            return collect(bitsc[pl.ds(base, 16)], base, cnt_v)

        @plsc.parallel_loop(NSUB // 16, NCHUNK, unroll=4, carry=sweep_a)
        def mainsweep(c, cnt_v):
            base = c * 16
            return collect(dist2(base), base, cnt_v)

        m_tot = jnp.max(mainsweep)
        nch = lax.shift_right_logical(m_tot + 15, 4)

        lo = jnp.int32(0)
        cb = jnp.int32(0)
        for shift in (23, 15, 7):
            need = K - cb
            lo_s = jnp.full((16,), lo)
            hi_s = jnp.full((16,), hi)

            def lvl(c, delta, lo_s=lo_s, hi_s=hi_s, shift=shift):
                b = bufb[pl.ds(c * 16, 16)]
                valid = (c * 16 + lane) < m_tot
                inr = (b >= lo_s) & (b < hi_s) & valid
                key = jnp.bitwise_and(lax.shift_right_logical(b, shift), 255)
                plsc.addupdate_scatter(hist2, [histidx(key)], delta, mask=inr)

            plsc.parallel_loop(0, nch, unroll=4)(
                functools.partial(lvl, delta=ones16))
            b2 = scan_first_bin(need)
            cb2 = count_below(b2)
            plsc.parallel_loop(0, nch, unroll=4)(
                functools.partial(lvl, delta=negones16))
            lo = lo + lax.shift_left(b2, shift)
            hi = lo + lax.shift_left(jnp.int32(1), shift)
            cb = cb + cb2

        for j in range(KPAD // 16):
            outb[pl.ds(j * 16, 16)] = inf_bits
            outi[pl.ds(j * 16, 16)] = zeros16

        lo_s = jnp.full((16,), lo)
        hi_s = jnp.full((16,), hi)

        @plsc.parallel_loop(0, nch, unroll=2, carry=(zeros16, zeros16))
        def coll(c, carry):
            cs_v, cm_v = carry
            b = bufb[pl.ds(c * 16, 16)]
            ix = bufi[pl.ds(c * 16, 16)]
            valid = (c * 16 + lane) < m_tot
            sure = (b < lo_s) & valid
            mid = (b >= lo_s) & (b < hi_s) & valid
            psure = cs_v + plsc.cumsum(jnp.where(sure, 1, 0)) - 1
            pmid = cb + cm_v + plsc.cumsum(jnp.where(mid, 1, 0)) - 1
            okm = mid & (pmid < K)
            plsc.store_scatter(outb, [psure], b, mask=sure)
            plsc.store_scatter(outi, [psure], ix, mask=sure)
            plsc.store_scatter(outb, [pmid], b, mask=okm)
            plsc.store_scatter(outi, [pmid], ix, mask=okm)
            return (cs_v + plsc.all_reduce_population_count(sure),
                    cm_v + plsc.all_reduce_population_count(mid))

        def minmax(ak, av, bk, bv):
            c = ak <= bk
            return (jnp.where(c, ak, bk), jnp.where(c, av, bv),
                    jnp.where(c, bk, ak), jnp.where(c, bv, av))

        def merge16(ak, av, bk, bv):
            rk = lax.rev(bk, (0,))
            rv = lax.rev(bv, (0,))
            lk, lv, hk, hv = minmax(ak, av, rk, rv)
            lk, lv = plsc.sort_key_val(lk, lv)
            hk, hv = plsc.sort_key_val(hk, hv)
            return lk, lv, hk, hv

        sk, sv = [], []
        for j in range(4):
            kj, vj = plsc.sort_key_val(outb[pl.ds(j * 16, 16)],
                                       outi[pl.ds(j * 16, 16)])
            sk.append(kj)
            sv.append(vj)
        a0k, a0v, a1k, a1v = merge16(sk[0], sv[0], sk[1], sv[1])
        b0k, b0v, b1k, b1v = merge16(sk[2], sv[2], sk[3], sv[3])
        rb0k, rb0v = lax.rev(b1k, (0,)), lax.rev(b1v, (0,))
        rb1k, rb1v = lax.rev(b0k, (0,)), lax.rev(b0v, (0,))
        l0k, l0v, h0k, h0v = minmax(a0k, a0v, rb0k, rb0v)
        l1k, l1v, h1k, h1v = minmax(a1k, a1v, rb1k, rb1v)
        p0k, p0v, p1k, p1v = minmax(l0k, l0v, l1k, l1v)
        q0k, q0v, q1k, q1v = minmax(h0k, h0v, h1k, h1v)
        f0k, f0v = plsc.sort_key_val(p0k, p0v)
        f1k, f1v = plsc.sort_key_val(p1k, p1v)
        f2k, f2v = plsc.sort_key_val(q0k, q0v)
        f3k, f3v = plsc.sort_key_val(q1k, q1v)

        sbase = jnp.bitwise_and(qi, STGQ - 1) * K
        for j, (fk, fv) in enumerate(((f0k, f0v), (f1k, f1v),
                                      (f2k, f2v))):
            spos = sbase + j * 16 + lane
            plsc.store_scatter(stg_i, [spos], fv)
            plsc.store_scatter(stg_d, [spos],
                               lax.bitcast_convert_type(fk, jnp.float32))
        tail = lane < 2
        tpos = sbase + 48 + lane
        plsc.store_scatter(stg_i, [tpos], f3v, mask=tail)
        plsc.store_scatter(stg_d, [tpos],
                           lax.bitcast_convert_type(f3k, jnp.float32),
                           mask=tail)

        @pl.when(jnp.bitwise_and(qi, STGQ - 1) == STGQ - 1)
        def _flush():
            hbase = pl.multiple_of((q - (STGQ - 1)) * K, STGQ * K)
            pltpu.sync_copy(stg_i, nbr_out.at[pl.ds(hbase, STGQ * K)])
            pltpu.sync_copy(stg_d, d2_out.at[pl.ds(hbase, STGQ * K)])

        return 0

    lax.fori_loop(0, QPW, per_query, 0)


@functools.partial(
    pl.kernel,
    out_type=(jax.ShapeDtypeStruct((N * K,), jnp.int32),
              jax.ShapeDtypeStruct((N * K,), jnp.float32)),
    mesh=plsc.VectorSubcoreMesh(core_axis_name="c", subcore_axis_name="s"),
    compiler_params=pltpu.CompilerParams(needs_layout_passes=False),
    scratch_types=[
        pltpu.VMEM((N,), jnp.float32),
        pltpu.VMEM((N,), jnp.float32),
        pltpu.VMEM((N,), jnp.float32),
        pltpu.VMEM((BUFCAP,), jnp.int32),
        pltpu.VMEM((BUFCAP,), jnp.int32),
        pltpu.VMEM((4096,), jnp.int32),
        pltpu.VMEM((256,), jnp.int32),
        pltpu.VMEM((NSUB,), jnp.int32),
        pltpu.VMEM((KPAD,), jnp.int32),
        pltpu.VMEM((KPAD,), jnp.int32),
        pltpu.VMEM((STGQ * K,), jnp.int32),
        pltpu.VMEM((STGQ * K,), jnp.float32),
        pltpu.SemaphoreType.DMA,
    ],
)
def _knn_sc(x_h, y_h, z_h, nbr_out, d2_out, *rest):
    _knn_sc_body(x_h, y_h, z_h, nbr_out, d2_out, *rest)


_RDF_BLK = 8192


def _rdf_kernel(d2_ref, out_ref):
    r = jnp.sqrt(d2_ref[...])
    mus = [MAX_DIST * i / (NUM_BINS - 1) for i in range(NUM_BINS)]
    rows = [jnp.exp(-GAMMA * (r - m) ** 2)[None, :] for m in mus]
    out_ref[...] = jnp.concatenate(rows, axis=0)


def _rdf5(d2):
    e = d2.shape[0]
    nblk = e // _RDF_BLK
    return pl.pallas_call(
        _rdf_kernel,
        grid=(2 * nblk,),
        in_specs=[pl.BlockSpec((_RDF_BLK,), lambda i: (i % nblk,))],
        out_specs=pl.BlockSpec((NUM_BINS, _RDF_BLK), lambda i: (0, i)),
        out_shape=jax.ShapeDtypeStruct((NUM_BINS, 2 * e), jnp.float32),
    )(d2)


_EI_ROWS = N * K // 128
_EI_RB = 400


def _edge_kernel(src_ref, out_ref):
    j = pl.program_id(0)
    s = src_ref[...]
    r_io = lax.broadcasted_iota(jnp.int32, (_EI_RB, 128), 0)
    c_io = lax.broadcasted_iota(jnp.int32, (_EI_RB, 128), 1)
    e = (j * _EI_RB + r_io) * 128 + c_io
    d = e // K
    out_ref[0, 0] = s
    out_ref[0, 1] = d
    out_ref[1, 0] = d
    out_ref[1, 1] = s


def _edge_index(src):
    out = pl.pallas_call(
        _edge_kernel,
        grid=(_EI_ROWS // _EI_RB,),
        in_specs=[pl.BlockSpec((_EI_RB, 128), lambda j: (j, 0))],
        out_specs=pl.BlockSpec((2, 2, _EI_RB, 128), lambda j: (0, 0, j, 0)),
        out_shape=jax.ShapeDtypeStruct((2, 2, _EI_ROWS, 128), jnp.int32),
    )(src.reshape(_EI_ROWS, 128))
    return out.reshape(2, 2 * N * K)


def kernel(pos):
    nbr50, d2_50 = _knn_sc(pos[:, 0], pos[:, 1], pos[:, 2])
    rdf = _rdf5(d2_50).T
    edge_index = _edge_index(nbr50)
    return edge_index, rdf


# speedup vs baseline: 1.0342x; 1.0176x over previous
import functools

import jax
import jax.numpy as jnp
from jax import lax
from jax.experimental import pallas as pl
from jax.experimental.pallas import tpu as pltpu
from jax.experimental.pallas import tpu_sc as plsc

N = 16384
K = 50
KPAD = 64
NUM_BINS = 5
MAX_DIST = 10.0
GAMMA = 0.5

NC = 2
NS = 16
NW = NC * NS
QPW = N // NW
NCHUNK = N // 16
NSUB = 4096
BUFCAP = 4096
STGQ = 32
HIST_FLAT = False


def _knn_sc_body(x_h, y_h, z_h, nbr_out, d2_out,
                 x_v, y_v, z_v, bufb, bufi, hist2, tot_v, bitsc,
                 outb, outi, stg_i, stg_d, sem):
    lane = lax.iota(jnp.int32, 16)
    lane256 = lane * 256
    zeros16 = jnp.zeros((16,), jnp.int32)
    ones16 = jnp.ones((16,), jnp.int32)
    negones16 = -ones16
    inf_bits = jnp.full((16,), 0x7F800000, jnp.int32)
    nhist = 16 if HIST_FLAT else 256

    def histidx(key):
        return key if HIST_FLAT else lane256 + key

    def scan_first_bin(need, nbins=256):
        def g(gi, carry):
            cum, nb_v = carry
            acc = zeros16
            for l in range(16):
                acc = acc + hist2[pl.ds(l * nbins + gi * 16, 16)]
            tot_v[pl.ds(gi * 16, 16)] = acc
            cs = plsc.cumsum(acc) + cum
            nb_v = nb_v + plsc.all_reduce_population_count(cs < need)
            return jnp.max(cs), nb_v

        _, nb_v = lax.fori_loop(0, nbins // 16, g, (jnp.int32(0), zeros16))
        return jnp.max(nb_v)

    def count_below(b):
        def g(gi, cb_v):
            v = tot_v[pl.ds(gi * 16, 16)]
            return cb_v + jnp.where(gi * 16 + lane < b, v, 0)

        return jnp.sum(lax.fori_loop(0, 16, g, zeros16))

    wid = lax.axis_index("s") * NC + lax.axis_index("c")
    pltpu.sync_copy(x_h, x_v)
    pltpu.sync_copy(y_h, y_v)
    pltpu.sync_copy(z_h, z_v)

    @plsc.parallel_loop(0, nhist, unroll=4)
    def _clr(i):
        hist2[pl.ds(i * 16, 16)] = zeros16

    def per_query(qi, _):
        q = wid * QPW + qi
        qsplat = jnp.full((16,), q)
        qx = plsc.load_gather(x_v, [qsplat])
        qy = plsc.load_gather(y_v, [qsplat])
        qz = plsc.load_gather(z_v, [qsplat])

        def dist2(base):
            dx = x_v[pl.ds(base, 16)] - qx
            dy = y_v[pl.ds(base, 16)] - qy
            dz = z_v[pl.ds(base, 16)] - qz
            d2 = dx * dx + dy * dy + dz * dz
            return lax.bitcast_convert_type(d2, jnp.int32)

        @plsc.parallel_loop(0, NSUB // 16, unroll=8)
        def _sub_hist(c):
            bits = dist2(c * 16)
            bitsc[pl.ds(c * 16, 16)] = bits
            key = lax.shift_right_logical(bits, 22)
            plsc.addupdate_scatter(hist2, [lane * 512 + key], ones16)
        b1s = scan_first_bin(K + 1, nbins=512)
        hi = lax.shift_left(jnp.minimum(b1s + 1, 511), 22)
        hi_splat = jnp.full((16,), hi)

        @plsc.parallel_loop(0, 512, unroll=8)
        def _clr_q(i):
            hist2[pl.ds(i * 16, 16)] = zeros16

        def collect(bits, base, cnt_v):
            idxv = base + lane
            keep = (bits < hi_splat) & (idxv != qsplat)
            pos = cnt_v + plsc.cumsum(jnp.where(keep, 1, 0)) - 1
            okm = keep & (pos < BUFCAP)
            plsc.store_scatter(bufb, [pos], bits, mask=okm)
            plsc.store_scatter(bufi, [pos], idxv, mask=okm)
            return cnt_v + plsc.all_reduce_population_count(keep)

        @plsc.parallel_loop(0, NSUB // 16, unroll=4, carry=zeros16)
        def sweep_a(c, cnt_v):
            base = c * 16
            return collect(bitsc[pl.ds(base, 16)], base, cnt_v)

        @plsc.parallel_loop(NSUB // 16, NCHUNK, unroll=4, carry=sweep_a)
        def mainsweep(c, cnt_v):
            base = c * 16
            return collect(dist2(base), base, cnt_v)

        m_tot = jnp.max(mainsweep)
        nch = lax.shift_right_logical(m_tot + 15, 4)

        lo = jnp.int32(0)
        cb = jnp.int32(0)
        for shift in (23, 15, 7):
            need = K - cb
            lo_s = jnp.full((16,), lo)
            hi_s = jnp.full((16,), hi)

            def lvl(c, delta, lo_s=lo_s, hi_s=hi_s, shift=shift):
                b = bufb[pl.ds(c * 16, 16)]
                valid = (c * 16 + lane) < m_tot
                inr = (b >= lo_s) & (b < hi_s) & valid
                key = jnp.bitwise_and(lax.shift_right_logical(b, shift), 255)
                plsc.addupdate_scatter(hist2, [histidx(key)], delta, mask=inr)

            plsc.parallel_loop(0, nch, unroll=4)(
                functools.partial(lvl, delta=ones16))
            b2 = scan_first_bin(need)
            cb2 = count_below(b2)
            plsc.parallel_loop(0, nch, unroll=4)(
                functools.partial(lvl, delta=negones16))
            lo = lo + lax.shift_left(b2, shift)
            hi = lo + lax.shift_left(jnp.int32(1), shift)
            cb = cb + cb2

        for j in range(KPAD // 16):
            outb[pl.ds(j * 16, 16)] = inf_bits
            outi[pl.ds(j * 16, 16)] = zeros16

        lo_s = jnp.full((16,), lo)
        hi_s = jnp.full((16,), hi)

        @plsc.parallel_loop(0, nch, unroll=2, carry=(zeros16, zeros16))
        def coll(c, carry):
            cs_v, cm_v = carry
            b = bufb[pl.ds(c * 16, 16)]
            ix = bufi[pl.ds(c * 16, 16)]
            valid = (c * 16 + lane) < m_tot
            sure = (b < lo_s) & valid
            mid = (b >= lo_s) & (b < hi_s) & valid
            psure = cs_v + plsc.cumsum(jnp.where(sure, 1, 0)) - 1
            pmid = cb + cm_v + plsc.cumsum(jnp.where(mid, 1, 0)) - 1
            okm = mid & (pmid < K)
            plsc.store_scatter(outb, [psure], b, mask=sure)
            plsc.store_scatter(outi, [psure], ix, mask=sure)
            plsc.store_scatter(outb, [pmid], b, mask=okm)
            plsc.store_scatter(outi, [pmid], ix, mask=okm)
            return (cs_v + plsc.all_reduce_population_count(sure),
                    cm_v + plsc.all_reduce_population_count(mid))

        def minmax(ak, av, bk, bv):
            c = ak <= bk
            return (jnp.where(c, ak, bk), jnp.where(c, av, bv),
                    jnp.where(c, bk, ak), jnp.where(c, bv, av))

        def merge16(ak, av, bk, bv):
            rk = lax.rev(bk, (0,))
            rv = lax.rev(bv, (0,))
            lk, lv, hk, hv = minmax(ak, av, rk, rv)
            lk, lv = plsc.sort_key_val(lk, lv)
            hk, hv = plsc.sort_key_val(hk, hv)
            return lk, lv, hk, hv

        sk, sv = [], []
        for j in range(4):
            kj, vj = plsc.sort_key_val(outb[pl.ds(j * 16, 16)],
                                       outi[pl.ds(j * 16, 16)])
            sk.append(kj)
            sv.append(vj)
        a0k, a0v, a1k, a1v = merge16(sk[0], sv[0], sk[1], sv[1])
        b0k, b0v, b1k, b1v = merge16(sk[2], sv[2], sk[3], sv[3])
        rb0k, rb0v = lax.rev(b1k, (0,)), lax.rev(b1v, (0,))
        rb1k, rb1v = lax.rev(b0k, (0,)), lax.rev(b0v, (0,))
        l0k, l0v, h0k, h0v = minmax(a0k, a0v, rb0k, rb0v)
        l1k, l1v, h1k, h1v = minmax(a1k, a1v, rb1k, rb1v)
        p0k, p0v, p1k, p1v = minmax(l0k, l0v, l1k, l1v)
        q0k, q0v, q1k, q1v = minmax(h0k, h0v, h1k, h1v)
        f0k, f0v = plsc.sort_key_val(p0k, p0v)
        f1k, f1v = plsc.sort_key_val(p1k, p1v)
        f2k, f2v = plsc.sort_key_val(q0k, q0v)
        f3k, f3v = plsc.sort_key_val(q1k, q1v)

        sbase = jnp.bitwise_and(qi, STGQ - 1) * K
        for j, (fk, fv) in enumerate(((f0k, f0v), (f1k, f1v),
                                      (f2k, f2v))):
            spos = sbase + j * 16 + lane
            plsc.store_scatter(stg_i, [spos], fv)
            plsc.store_scatter(stg_d, [spos],
                               lax.bitcast_convert_type(fk, jnp.float32))
        tail = lane < 2
        tpos = sbase + 48 + lane
        plsc.store_scatter(stg_i, [tpos], f3v, mask=tail)
        plsc.store_scatter(stg_d, [tpos],
                           lax.bitcast_convert_type(f3k, jnp.float32),
                           mask=tail)

        @pl.when(jnp.bitwise_and(qi, STGQ - 1) == STGQ - 1)
        def _flush():
            hbase = pl.multiple_of((q - (STGQ - 1)) * K, STGQ * K)
            pltpu.sync_copy(stg_i, nbr_out.at[pl.ds(hbase, STGQ * K)])
            pltpu.sync_copy(stg_d, d2_out.at[pl.ds(hbase, STGQ * K)])

        return 0

    lax.fori_loop(0, QPW, per_query, 0)


@functools.partial(
    pl.kernel,
    out_type=(jax.ShapeDtypeStruct((N * K,), jnp.int32),
              jax.ShapeDtypeStruct((N * K,), jnp.float32)),
    mesh=plsc.VectorSubcoreMesh(core_axis_name="c", subcore_axis_name="s"),
    compiler_params=pltpu.CompilerParams(needs_layout_passes=False),
    scratch_types=[
        pltpu.VMEM((N,), jnp.float32),
        pltpu.VMEM((N,), jnp.float32),
        pltpu.VMEM((N,), jnp.float32),
        pltpu.VMEM((BUFCAP,), jnp.int32),
        pltpu.VMEM((BUFCAP,), jnp.int32),
        pltpu.VMEM((8192,), jnp.int32),
        pltpu.VMEM((512,), jnp.int32),
        pltpu.VMEM((NSUB,), jnp.int32),
        pltpu.VMEM((KPAD,), jnp.int32),
        pltpu.VMEM((KPAD,), jnp.int32),
        pltpu.VMEM((STGQ * K,), jnp.int32),
        pltpu.VMEM((STGQ * K,), jnp.float32),
        pltpu.SemaphoreType.DMA,
    ],
)
def _knn_sc(x_h, y_h, z_h, nbr_out, d2_out, *rest):
    _knn_sc_body(x_h, y_h, z_h, nbr_out, d2_out, *rest)


_RDF_BLK = 8192


def _rdf_kernel(d2_ref, out_ref):
    r = jnp.sqrt(d2_ref[...])
    mus = [MAX_DIST * i / (NUM_BINS - 1) for i in range(NUM_BINS)]
    rows = [jnp.exp(-GAMMA * (r - m) ** 2)[None, :] for m in mus]
    out_ref[...] = jnp.concatenate(rows, axis=0)


def _rdf5(d2):
    e = d2.shape[0]
    nblk = e // _RDF_BLK
    return pl.pallas_call(
        _rdf_kernel,
        grid=(2 * nblk,),
        in_specs=[pl.BlockSpec((_RDF_BLK,), lambda i: (i % nblk,))],
        out_specs=pl.BlockSpec((NUM_BINS, _RDF_BLK), lambda i: (0, i)),
        out_shape=jax.ShapeDtypeStruct((NUM_BINS, 2 * e), jnp.float32),
    )(d2)


_EI_ROWS = N * K // 128
_EI_RB = 400


def _edge_kernel(src_ref, out_ref):
    j = pl.program_id(0)
    s = src_ref[...]
    r_io = lax.broadcasted_iota(jnp.int32, (_EI_RB, 128), 0)
    c_io = lax.broadcasted_iota(jnp.int32, (_EI_RB, 128), 1)
    e = (j * _EI_RB + r_io) * 128 + c_io
    d = e // K
    out_ref[0, 0] = s
    out_ref[0, 1] = d
    out_ref[1, 0] = d
    out_ref[1, 1] = s


def _edge_index(src):
    out = pl.pallas_call(
        _edge_kernel,
        grid=(_EI_ROWS // _EI_RB,),
        in_specs=[pl.BlockSpec((_EI_RB, 128), lambda j: (j, 0))],
        out_specs=pl.BlockSpec((2, 2, _EI_RB, 128), lambda j: (0, 0, j, 0)),
        out_shape=jax.ShapeDtypeStruct((2, 2, _EI_ROWS, 128), jnp.int32),
    )(src.reshape(_EI_ROWS, 128))
    return out.reshape(2, 2 * N * K)


def kernel(pos):
    nbr50, d2_50 = _knn_sc(pos[:, 0], pos[:, 1], pos[:, 2])
    rdf = _rdf5(d2_50).T
    edge_index = _edge_index(nbr50)
    return edge_index, rdf
